# baseline (speedup 1.0000x reference)
import functools

import jax
import jax.numpy as jnp
from jax import lax
from jax.experimental import pallas as pl
from jax.experimental.pallas import tpu as pltpu
from jax.experimental.pallas import tpu_sc as plsc

T, D, S = 16384, 2048, 16
T_TILE = 256
NSTREAM = 8
NCHUNK = T // T_TILE
NW = 32
CPW = NCHUNK // NW
NGRP = T_TILE // 16


def _tc_body(*refs):
    x_refs = refs[:NSTREAM]
    g_refs = refs[NSTREAM:2 * NSTREAM]
    wc_ref, batt_ref, bstrat_ref, out_ref = refs[2 * NSTREAM:]
    wc = wc_ref[...]
    batt = batt_ref[...]
    bstrat = bstrat_ref[...]
    for j in range(NSTREAM):
        acc = jnp.dot(x_refs[j][...], wc, preferred_element_type=jnp.float32)
        z = acc[:, :S] + batt + g_refs[j][...]
        sig = acc[:, S:] + bstrat
        out_ref[j] = jnp.concatenate([z, sig], axis=1).T


def _sc_body(zst_hbm, out_hbm, zv, outv):
    wid = lax.axis_index("s") * 2 + lax.axis_index("c")
    for k in range(CPW):
        chunk = wid * CPW + k
        pltpu.sync_copy(zst_hbm.at[chunk], zv)

        def body(g, carry):
            base = g * 16
            zs = [zv[s, pl.ds(base, 16)] for s in range(S)]
            m = zs[0]
            for s in range(1, S):
                m = jnp.maximum(m, zs[s])
            num = jnp.zeros((16,), jnp.float32)
            den = jnp.zeros((16,), jnp.float32)
            for s in range(S):
                e = jnp.exp(zs[s] - m)
                den = den + e
                num = num + e * zv[S + s, pl.ds(base, 16)]
            outv[pl.ds(base, 16)] = num / den
            return carry

        lax.fori_loop(0, NGRP, body, 0)
        pltpu.sync_copy(outv, out_hbm.at[pl.ds(chunk * T_TILE, T_TILE)])


@jax.jit
def kernel(x, gumbel_noise, W_att, b_att, W_strat, b_strat, adaptive_bias):
    Wc = jnp.concatenate([W_att, W_strat[:, :, 0].T], axis=1)
    batt = (b_att + adaptive_bias).reshape(1, S)
    bstrat = b_strat[:, 0].reshape(1, S)
    grid = (T // (NSTREAM * T_TILE),)

    def xmap(j):
        return lambda i: (NSTREAM * i + j, 0)

    zst = pl.pallas_call(
        _tc_body,
        grid=grid,
        in_specs=(
            [pl.BlockSpec((T_TILE, D), xmap(j)) for j in range(NSTREAM)]
            + [pl.BlockSpec((T_TILE, S), xmap(j)) for j in range(NSTREAM)]
            + [
                pl.BlockSpec((D, 2 * S), lambda i: (0, 0)),
                pl.BlockSpec((1, S), lambda i: (0, 0)),
                pl.BlockSpec((1, S), lambda i: (0, 0)),
            ]
        ),
        out_specs=pl.BlockSpec((NSTREAM, 2 * S, T_TILE), lambda i: (i, 0, 0)),
        out_shape=jax.ShapeDtypeStruct((NCHUNK, 2 * S, T_TILE), jnp.float32),
    )(*([x] * NSTREAM + [gumbel_noise] * NSTREAM + [Wc, batt, bstrat]))

    gate = functools.partial(
        pl.kernel,
        mesh=plsc.VectorSubcoreMesh(core_axis_name="c", subcore_axis_name="s"),
        out_type=jax.ShapeDtypeStruct((T,), jnp.float32),
        scratch_types=[
            pltpu.VMEM((2 * S, T_TILE), jnp.float32),
            pltpu.VMEM((T_TILE,), jnp.float32),
        ],
    )(_sc_body)
    out = gate(zst)
    return out.reshape(T, 1)
